# baseline (speedup 1.0000x reference)
import functools

import jax
import jax.numpy as jnp
from jax import lax
from jax.experimental import pallas as pl
from jax.experimental.pallas import tpu as pltpu
from jax.experimental.pallas import tpu_sc as plsc

NUM_CLASSES = 80
SOFT_CENTER_RADIUS = 3.0
TOPK = 13
IOU_WEIGHT = 3.0
INF = 100000000.0
EPS = 1e-7
BIG = 3.0e38
LANES = 16


def _cost_kernel(pb_ref, ps_ref, pr_ref, gt_ref, lab_ref, flag_ref,
                 cost_out, iou_out, amin_out):
    N = pb_ref.shape[2]
    G = gt_ref.shape[1]

    pb = pb_ref[0]
    pr = pr_ref[...]
    gt = gt_ref[0]
    lab = lab_ref[0]
    flag = flag_ref[0]

    px = pr[0:1, :]
    py = pr[1:2, :]
    pstride = pr[2:3, :]
    x1 = pb[0:1, :]
    y1 = pb[1:2, :]
    x2 = pb[2:3, :]
    y2 = pb[3:4, :]
    gx1 = gt[:, 0:1]
    gy1 = gt[:, 1:2]
    gx2 = gt[:, 2:3]
    gy2 = gt[:, 3:4]

    in_gts = (px > gx1) & (py > gy1) & (px < gx2) & (py < gy2) & (flag > 0)
    valid = jnp.sum(in_gts.astype(jnp.float32), axis=0, keepdims=True) > 0
    validf = valid.astype(jnp.float32)

    gcx = (gx1 + gx2) * 0.5
    gcy = (gy1 + gy2) * 0.5
    dist = jnp.sqrt((px - gcx) ** 2 + (py - gcy) ** 2) / pstride
    dist = dist * validf
    soft = jnp.power(10.0, dist - SOFT_CENTER_RADIUS)

    iw = jnp.maximum(jnp.minimum(x2, gx2) - jnp.maximum(x1, gx1), 0.0)
    ih = jnp.maximum(jnp.minimum(y2, gy2) - jnp.maximum(y1, gy1), 0.0)
    overlap = iw * ih
    area1 = (x2 - x1) * (y2 - y1)
    area2 = (gx2 - gx1) * (gy2 - gy1)
    union = jnp.maximum(area1 + area2 - overlap, 1e-6)
    iou = overlap / union
    iou_cost = -jnp.log(iou + EPS) * IOU_WEIGHT

    sc = ps_ref[0]
    idx8 = jnp.broadcast_to(lab & 7, (G, N))
    labblk = lab >> 3
    x = jnp.zeros((G, N), jnp.float32)
    for blk in range(NUM_CLASSES // 8):
        xb = jnp.take_along_axis(sc[blk * 8:(blk + 1) * 8, :], idx8, axis=0)
        x = jnp.where(labblk == blk, xb, x)
    sig = jax.nn.sigmoid(x)
    bce = jnp.maximum(x, 0.0) - x * iou + jnp.log1p(jnp.exp(-jnp.abs(x)))
    cost = bce * (iou - sig) ** 2 + iou_cost + soft
    cost = jnp.where(valid, cost, INF)

    gidx = lax.broadcasted_iota(jnp.int32, (G, N), 0)
    rmin = jnp.min(cost, axis=0, keepdims=True)
    amin = jnp.min(jnp.where(cost == rmin, gidx, G), axis=0, keepdims=True)

    cost_out[0] = cost
    iou_out[0] = iou
    amin_out[0] = amin


def _sc_topk(cost_hbm, iou_hbm, idx_hbm, cost_v, iou_v, out_v,
             best_v, bidx_v, thr_s):
    ncols = cost_hbm.shape[0]
    n = cost_hbm.shape[1]
    nch = n // LANES
    nw = 32
    cols_per_w = ncols // nw
    wid = lax.axis_index("s") * 2 + lax.axis_index("c")
    base = wid * cols_per_w

    iota16 = lax.broadcasted_iota(jnp.int32, (LANES,), 0)
    zeros16 = jnp.zeros((LANES,), jnp.int32)
    last16 = jnp.full((LANES,), LANES - 1, jnp.int32)
    rots = [(iota16 + (1 << s)) % LANES for s in range(4)]

    def splat(vec, idx):
        return jnp.take_along_axis(vec, idx, axis=0)

    def lane_min(v):
        for r in rots:
            v = jnp.minimum(v, splat(v, r))
        return v[0]

    def lane_max(v):
        for r in rots:
            v = jnp.maximum(v, splat(v, r))
        return v[0]

    def col_body(r, carry):
        row = base + r
        pltpu.sync_copy(cost_hbm.at[row], cost_v)
        pltpu.sync_copy(iou_hbm.at[row], iou_v)

        best_v[...] = jnp.full((LANES,), -1.0, jnp.float32)
        thr_s[0] = -1.0

        def iou_chunk(i, c):
            v = iou_v[pl.ds(i * LANES, LANES)]

            @pl.when(lane_max(v) > thr_s[0])
            def _():
                s = lax.sort(v, dimension=0)
                rb = lax.rev(best_v[...], (0,))
                keep = jnp.maximum(s, rb)
                nb = lax.sort(keep, dimension=0)
                best_v[...] = nb
                thr_s[0] = nb[0]

            return c

        lax.fori_loop(0, nch, iou_chunk, 0)
        ibest = best_v[...]

        def sum_body(j, acc):
            return acc + splat(ibest, last16 - j)

        s13 = lax.fori_loop(0, TOPK, sum_body, jnp.zeros((LANES,), jnp.float32))
        k = jnp.maximum(s13.astype(jnp.int32), 1)

        best_v[...] = jnp.full((LANES,), BIG, jnp.float32)
        bidx_v[...] = jnp.full((LANES,), -1, jnp.int32)
        thr_s[0] = BIG

        def cost_chunk(i, c):
            v = cost_v[pl.ds(i * LANES, LANES)]

            @pl.when(lane_min(v) < thr_s[0])
            def _():
                vidx = iota16 + i * LANES
                s, si = plsc.sort_key_val(v, vidx)
                rb = lax.rev(best_v[...], (0,))
                rbi = lax.rev(bidx_v[...], (0,))
                take = s <= rb
                ck = jnp.where(take, s, rb)
                ci = jnp.where(take, si, rbi)
                nb, nbi = plsc.sort_key_val(ck, ci)
                best_v[...] = nb
                bidx_v[...] = nbi
                thr_s[0] = nb[LANES - 1]

            return c

        lax.fori_loop(0, nch, cost_chunk, 0)

        out_v[...] = jnp.where(iota16 < k, bidx_v[...], -1)
        pltpu.sync_copy(out_v, idx_hbm.at[row])
        return carry

    lax.fori_loop(0, cols_per_w, col_body, 0)


def _assemble_kernel(iou_ref, idx_ref, amin_ref, gt_ref, lab_ref, flag_ref,
                     lab_out, bbox_out, met_out):
    N = iou_ref.shape[2]
    G = iou_ref.shape[1]

    iou = iou_ref[0]
    sel_idx = idx_ref[0]
    amin = amin_ref[0]
    gt = gt_ref[0]
    lab = lab_ref[0]
    flag = flag_ref[0]

    pidx = lax.broadcasted_iota(jnp.int32, (G, N), 1)
    gt_ok = flag > 0

    matching = jnp.zeros((G, N), jnp.float32)
    for j in range(TOPK):
        hit = (pidx == sel_idx[:, j:j + 1]) & gt_ok
        matching = jnp.where(hit, 1.0, matching)

    cnt = jnp.sum(matching, axis=0, keepdims=True)
    gidx = lax.broadcasted_iota(jnp.int32, (G, N), 0)
    fmatch = jnp.min(jnp.where(matching > 0, gidx, G), axis=0, keepdims=True)
    mg = jnp.where(cnt > 1, amin, fmatch)
    fg = cnt > 0

    sel = (gidx == mg).astype(jnp.float32)
    met = jnp.sum(sel * iou, axis=0, keepdims=True)
    labf = jnp.sum(sel * lab.astype(jnp.float32), axis=0, keepdims=True)
    bbox = jnp.concatenate(
        [jnp.sum(sel * gt[:, c:c + 1], axis=0, keepdims=True)
         for c in range(4)], axis=0)

    lab_out[0] = jnp.where(fg, labf.astype(jnp.int32), NUM_CLASSES)
    met_out[0] = jnp.where(fg, met, 0.0)
    bbox_out[0] = jnp.where(fg, bbox, 0.0)


def kernel(pred_bboxes, pred_scores, priors, gt_labels, gt_bboxes, pad_bbox_flag):
    B, N, _ = pred_bboxes.shape
    G = gt_bboxes.shape[1]
    pb_t = jnp.transpose(pred_bboxes, (0, 2, 1))
    ps_t = jnp.transpose(pred_scores, (0, 2, 1))
    pr_t = jnp.transpose(priors, (1, 0))
    lab = gt_labels.astype(jnp.int32)

    cost, iou, amin = pl.pallas_call(
        _cost_kernel,
        grid=(B,),
        in_specs=[
            pl.BlockSpec((1, 4, N), lambda b: (b, 0, 0)),
            pl.BlockSpec((1, NUM_CLASSES, N), lambda b: (b, 0, 0)),
            pl.BlockSpec((4, N), lambda b: (0, 0)),
            pl.BlockSpec((1, G, 4), lambda b: (b, 0, 0)),
            pl.BlockSpec((1, G, 1), lambda b: (b, 0, 0)),
            pl.BlockSpec((1, G, 1), lambda b: (b, 0, 0)),
        ],
        out_specs=[
            pl.BlockSpec((1, G, N), lambda b: (b, 0, 0)),
            pl.BlockSpec((1, G, N), lambda b: (b, 0, 0)),
            pl.BlockSpec((1, 1, N), lambda b: (b, 0, 0)),
        ],
        out_shape=[
            jax.ShapeDtypeStruct((B, G, N), jnp.float32),
            jax.ShapeDtypeStruct((B, G, N), jnp.float32),
            jax.ShapeDtypeStruct((B, 1, N), jnp.int32),
        ],
    )(pb_t, ps_t, pr_t, gt_bboxes, lab, pad_bbox_flag)

    sc_topk = functools.partial(
        pl.kernel,
        out_type=jax.ShapeDtypeStruct((B * G, LANES), jnp.int32),
        mesh=plsc.VectorSubcoreMesh(core_axis_name="c", subcore_axis_name="s"),
        compiler_params=pltpu.CompilerParams(needs_layout_passes=False),
        scratch_types=[
            pltpu.VMEM((N,), jnp.float32),
            pltpu.VMEM((N,), jnp.float32),
            pltpu.VMEM((LANES,), jnp.int32),
            pltpu.VMEM((LANES,), jnp.float32),
            pltpu.VMEM((LANES,), jnp.int32),
            pltpu.SMEM((1,), jnp.float32),
        ],
    )(_sc_topk)
    sel_idx = sc_topk(cost.reshape(B * G, N), iou.reshape(B * G, N))

    labels, bboxes_t, metrics = pl.pallas_call(
        _assemble_kernel,
        grid=(B,),
        in_specs=[
            pl.BlockSpec((1, G, N), lambda b: (b, 0, 0)),
            pl.BlockSpec((1, G, LANES), lambda b: (b, 0, 0)),
            pl.BlockSpec((1, 1, N), lambda b: (b, 0, 0)),
            pl.BlockSpec((1, G, 4), lambda b: (b, 0, 0)),
            pl.BlockSpec((1, G, 1), lambda b: (b, 0, 0)),
            pl.BlockSpec((1, G, 1), lambda b: (b, 0, 0)),
        ],
        out_specs=[
            pl.BlockSpec((1, 1, N), lambda b: (b, 0, 0)),
            pl.BlockSpec((1, 4, N), lambda b: (b, 0, 0)),
            pl.BlockSpec((1, 1, N), lambda b: (b, 0, 0)),
        ],
        out_shape=[
            jax.ShapeDtypeStruct((B, 1, N), jnp.int32),
            jax.ShapeDtypeStruct((B, 4, N), jnp.float32),
            jax.ShapeDtypeStruct((B, 1, N), jnp.float32),
        ],
    )(iou, sel_idx.reshape(B, G, LANES), amin, gt_bboxes, lab, pad_bbox_flag)

    weights = jnp.ones((B, N), dtype=gt_bboxes.dtype)
    bboxes = jnp.transpose(bboxes_t, (0, 2, 1))
    return labels[:, 0], weights, bboxes, metrics[:, 0]


# speedup vs baseline: 2.3519x; 2.3519x over previous
import functools

import jax
import jax.numpy as jnp
from jax import lax
from jax.experimental import pallas as pl
from jax.experimental.pallas import tpu as pltpu
from jax.experimental.pallas import tpu_sc as plsc

NUM_CLASSES = 80
SOFT_CENTER_RADIUS = 3.0
TOPK = 13
IOU_WEIGHT = 3.0
INF = 100000000.0
EPS = 1e-7
BIG = 3.0e38
LANES = 16


def _cost_kernel(pb_ref, ps_ref, pr_ref, gt_ref, lab_ref, flag_ref,
                 cost_out, iou_out, amin_out):
    N = pb_ref.shape[2]
    G = gt_ref.shape[1]

    pb = pb_ref[0]
    pr = pr_ref[...]
    gt = gt_ref[0]
    lab = lab_ref[0]
    flag = flag_ref[0]

    px = pr[0:1, :]
    py = pr[1:2, :]
    pstride = pr[2:3, :]
    x1 = pb[0:1, :]
    y1 = pb[1:2, :]
    x2 = pb[2:3, :]
    y2 = pb[3:4, :]
    gx1 = gt[:, 0:1]
    gy1 = gt[:, 1:2]
    gx2 = gt[:, 2:3]
    gy2 = gt[:, 3:4]

    in_gts = (px > gx1) & (py > gy1) & (px < gx2) & (py < gy2) & (flag > 0)
    valid = jnp.sum(in_gts.astype(jnp.float32), axis=0, keepdims=True) > 0
    validf = valid.astype(jnp.float32)

    gcx = (gx1 + gx2) * 0.5
    gcy = (gy1 + gy2) * 0.5
    dist = jnp.sqrt((px - gcx) ** 2 + (py - gcy) ** 2) / pstride
    dist = dist * validf
    soft = jnp.power(10.0, dist - SOFT_CENTER_RADIUS)

    iw = jnp.maximum(jnp.minimum(x2, gx2) - jnp.maximum(x1, gx1), 0.0)
    ih = jnp.maximum(jnp.minimum(y2, gy2) - jnp.maximum(y1, gy1), 0.0)
    overlap = iw * ih
    area1 = (x2 - x1) * (y2 - y1)
    area2 = (gx2 - gx1) * (gy2 - gy1)
    union = jnp.maximum(area1 + area2 - overlap, 1e-6)
    iou = overlap / union
    iou_cost = -jnp.log(iou + EPS) * IOU_WEIGHT

    sc = ps_ref[0]
    idx8 = jnp.broadcast_to(lab & 7, (G, N))
    labblk = lab >> 3
    x = jnp.zeros((G, N), jnp.float32)
    for blk in range(NUM_CLASSES // 8):
        xb = jnp.take_along_axis(sc[blk * 8:(blk + 1) * 8, :], idx8, axis=0)
        x = jnp.where(labblk == blk, xb, x)
    sig = jax.nn.sigmoid(x)
    bce = jnp.maximum(x, 0.0) - x * iou + jnp.log1p(jnp.exp(-jnp.abs(x)))
    cost = bce * (iou - sig) ** 2 + iou_cost + soft
    cost = jnp.where(valid, cost, INF)

    gidx = lax.broadcasted_iota(jnp.int32, (G, N), 0)
    rmin = jnp.min(cost, axis=0, keepdims=True)
    amin = jnp.min(jnp.where(cost == rmin, gidx, G), axis=0, keepdims=True)

    cost_out[0] = cost
    iou_out[0] = iou
    amin_out[0] = amin


def _sc_topk(cost_hbm, iou_hbm, idx_hbm, cost_v, iou_v, out_v,
             gm_v, best_v, bidx_v, sem):
    ncols = cost_hbm.shape[0]
    n = cost_hbm.shape[1]
    nch = n // LANES
    grp = 15
    ngrp = nch // grp
    nw = 32
    cols_per_w = ncols // nw
    wid = lax.axis_index("s") * 2 + lax.axis_index("c")
    base = wid * cols_per_w

    iota16 = lax.broadcasted_iota(jnp.int32, (LANES,), 0)
    zeros16 = jnp.zeros((LANES,), jnp.int32)
    last16 = jnp.full((LANES,), LANES - 1, jnp.int32)
    rots = [(iota16 + (1 << s)) % LANES for s in range(4)]

    def splat(vec, idx):
        return jnp.take_along_axis(vec, idx, axis=0)

    def lane_min(v):
        for r in rots:
            v = jnp.minimum(v, splat(v, r))
        return v[0]

    def lane_max(v):
        for r in rots:
            v = jnp.maximum(v, splat(v, r))
        return v[0]

    def col_body(r, carry):
        row = base + r
        cin = pltpu.make_async_copy(cost_hbm.at[row], cost_v, sem)
        iin = pltpu.make_async_copy(iou_hbm.at[row], iou_v, sem)
        cin.start()
        iin.start()
        cin.wait()
        iin.wait()

        def iou_grp(gi, c):
            m = iou_v[pl.ds(gi * grp * LANES, LANES)]
            for j in range(1, grp):
                m = jnp.maximum(m, iou_v[pl.ds((gi * grp + j) * LANES, LANES)])
            gm_v[pl.ds(gi * LANES, LANES)] = m
            return c

        lax.fori_loop(0, ngrp, iou_grp, 0)

        def tmax_merge(gi, kept):
            s = lax.sort(gm_v[pl.ds(gi * LANES, LANES)], dimension=0)
            hi = jnp.maximum(s, lax.rev(kept, (0,)))
            return lax.sort(hi, dimension=0)

        kept = lax.fori_loop(0, ngrp, tmax_merge,
                             jnp.full((LANES,), -1.0, jnp.float32))
        t_iou = kept[3]

        best_v[...] = jnp.full((LANES,), -1.0, jnp.float32)

        def iou_passb(gi, c):
            gmv = gm_v[pl.ds(gi * LANES, LANES)]

            @pl.when(lane_max(gmv) >= t_iou)
            def _():
                for j in range(grp):
                    v = iou_v[pl.ds((gi * grp + j) * LANES, LANES)]
                    s = lax.sort(v, dimension=0)
                    hi = jnp.maximum(s, lax.rev(best_v[...], (0,)))
                    best_v[...] = lax.sort(hi, dimension=0)

            return c

        lax.fori_loop(0, ngrp, iou_passb, 0)
        ibest = best_v[...]

        def sum_body(j, acc):
            return acc + splat(ibest, last16 - j)

        s13 = lax.fori_loop(0, TOPK, sum_body, jnp.zeros((LANES,), jnp.float32))
        k = jnp.maximum(s13.astype(jnp.int32), 1)

        def cost_grp(gi, c):
            m = cost_v[pl.ds(gi * grp * LANES, LANES)]
            for j in range(1, grp):
                m = jnp.minimum(m, cost_v[pl.ds((gi * grp + j) * LANES, LANES)])
            gm_v[pl.ds(gi * LANES, LANES)] = m
            return c

        lax.fori_loop(0, ngrp, cost_grp, 0)

        def tmin_merge(gi, kept):
            s = lax.sort(gm_v[pl.ds(gi * LANES, LANES)], dimension=0)
            lo = jnp.minimum(s, lax.rev(kept, (0,)))
            return lax.sort(lo, dimension=0)

        kept = lax.fori_loop(0, ngrp, tmin_merge,
                             jnp.full((LANES,), BIG, jnp.float32))
        t_cost = kept[TOPK - 1]

        best_v[...] = jnp.full((LANES,), BIG, jnp.float32)
        bidx_v[...] = jnp.full((LANES,), -1, jnp.int32)

        def cost_passb(gi, c):
            gmv = gm_v[pl.ds(gi * LANES, LANES)]

            @pl.when(lane_min(gmv) <= t_cost)
            def _():
                for j in range(grp):
                    ci = gi * grp + j
                    v = cost_v[pl.ds(ci * LANES, LANES)]
                    vidx = iota16 + ci * LANES
                    s, si = plsc.sort_key_val(v, vidx)
                    rb = lax.rev(best_v[...], (0,))
                    rbi = lax.rev(bidx_v[...], (0,))
                    take = s <= rb
                    nb, nbi = plsc.sort_key_val(jnp.where(take, s, rb),
                                                jnp.where(take, si, rbi))
                    best_v[...] = nb
                    bidx_v[...] = nbi

            return c

        lax.fori_loop(0, ngrp, cost_passb, 0)

        out_v[pl.ds(r * LANES, LANES)] = jnp.where(iota16 < k, bidx_v[...], -1)
        return carry

    lax.fori_loop(0, cols_per_w, col_body, 0)
    pltpu.sync_copy(out_v, idx_hbm.at[pl.ds(base * LANES, cols_per_w * LANES)])


def _assemble_kernel(iou_ref, idx_ref, amin_ref, gt_ref, lab_ref, flag_ref,
                     lab_out, bbox_out, met_out):
    N = iou_ref.shape[2]
    G = iou_ref.shape[1]

    iou = iou_ref[0]
    sel_idx = idx_ref[0]
    amin = amin_ref[0]
    gt = gt_ref[0]
    lab = lab_ref[0]
    flag = flag_ref[0]

    pidx = lax.broadcasted_iota(jnp.int32, (G, N), 1)
    gt_ok = flag > 0

    matching = jnp.zeros((G, N), jnp.float32)
    for j in range(TOPK):
        hit = (pidx == sel_idx[:, j:j + 1]) & gt_ok
        matching = jnp.where(hit, 1.0, matching)

    cnt = jnp.sum(matching, axis=0, keepdims=True)
    gidx = lax.broadcasted_iota(jnp.int32, (G, N), 0)
    fmatch = jnp.min(jnp.where(matching > 0, gidx, G), axis=0, keepdims=True)
    mg = jnp.where(cnt > 1, amin, fmatch)
    fg = cnt > 0

    sel = (gidx == mg).astype(jnp.float32)
    met = jnp.sum(sel * iou, axis=0, keepdims=True)
    labf = jnp.sum(sel * lab.astype(jnp.float32), axis=0, keepdims=True)
    bbox = jnp.concatenate(
        [jnp.sum(sel * gt[:, c:c + 1], axis=0, keepdims=True)
         for c in range(4)], axis=0)

    lab_out[0] = jnp.where(fg, labf.astype(jnp.int32), NUM_CLASSES)
    met_out[0] = jnp.where(fg, met, 0.0)
    bbox_out[0] = jnp.where(fg, bbox, 0.0)


def kernel(pred_bboxes, pred_scores, priors, gt_labels, gt_bboxes, pad_bbox_flag):
    B, N, _ = pred_bboxes.shape
    G = gt_bboxes.shape[1]
    pb_t = jnp.transpose(pred_bboxes, (0, 2, 1))
    ps_t = jnp.transpose(pred_scores, (0, 2, 1))
    pr_t = jnp.transpose(priors, (1, 0))
    lab = gt_labels.astype(jnp.int32)

    cost, iou, amin = pl.pallas_call(
        _cost_kernel,
        grid=(B,),
        in_specs=[
            pl.BlockSpec((1, 4, N), lambda b: (b, 0, 0)),
            pl.BlockSpec((1, NUM_CLASSES, N), lambda b: (b, 0, 0)),
            pl.BlockSpec((4, N), lambda b: (0, 0)),
            pl.BlockSpec((1, G, 4), lambda b: (b, 0, 0)),
            pl.BlockSpec((1, G, 1), lambda b: (b, 0, 0)),
            pl.BlockSpec((1, G, 1), lambda b: (b, 0, 0)),
        ],
        out_specs=[
            pl.BlockSpec((1, G, N), lambda b: (b, 0, 0)),
            pl.BlockSpec((1, G, N), lambda b: (b, 0, 0)),
            pl.BlockSpec((1, 1, N), lambda b: (b, 0, 0)),
        ],
        out_shape=[
            jax.ShapeDtypeStruct((B, G, N), jnp.float32),
            jax.ShapeDtypeStruct((B, G, N), jnp.float32),
            jax.ShapeDtypeStruct((B, 1, N), jnp.int32),
        ],
    )(pb_t, ps_t, pr_t, gt_bboxes, lab, pad_bbox_flag)

    ncols = B * G
    cols_per_w = ncols // 32
    ngrp = N // LANES // 15
    sc_topk = functools.partial(
        pl.kernel,
        out_type=jax.ShapeDtypeStruct((ncols * LANES,), jnp.int32),
        mesh=plsc.VectorSubcoreMesh(core_axis_name="c", subcore_axis_name="s"),
        compiler_params=pltpu.CompilerParams(needs_layout_passes=False),
        scratch_types=[
            pltpu.VMEM((N,), jnp.float32),
            pltpu.VMEM((N,), jnp.float32),
            pltpu.VMEM((cols_per_w * LANES,), jnp.int32),
            pltpu.VMEM((ngrp * LANES,), jnp.float32),
            pltpu.VMEM((LANES,), jnp.float32),
            pltpu.VMEM((LANES,), jnp.int32),
            pltpu.SemaphoreType.DMA,
        ],
    )(_sc_topk)
    sel_idx = sc_topk(cost.reshape(B * G, N), iou.reshape(B * G, N))

    labels, bboxes_t, metrics = pl.pallas_call(
        _assemble_kernel,
        grid=(B,),
        in_specs=[
            pl.BlockSpec((1, G, N), lambda b: (b, 0, 0)),
            pl.BlockSpec((1, G, LANES), lambda b: (b, 0, 0)),
            pl.BlockSpec((1, 1, N), lambda b: (b, 0, 0)),
            pl.BlockSpec((1, G, 4), lambda b: (b, 0, 0)),
            pl.BlockSpec((1, G, 1), lambda b: (b, 0, 0)),
            pl.BlockSpec((1, G, 1), lambda b: (b, 0, 0)),
        ],
        out_specs=[
            pl.BlockSpec((1, 1, N), lambda b: (b, 0, 0)),
            pl.BlockSpec((1, 4, N), lambda b: (b, 0, 0)),
            pl.BlockSpec((1, 1, N), lambda b: (b, 0, 0)),
        ],
        out_shape=[
            jax.ShapeDtypeStruct((B, 1, N), jnp.int32),
            jax.ShapeDtypeStruct((B, 4, N), jnp.float32),
            jax.ShapeDtypeStruct((B, 1, N), jnp.float32),
        ],
    )(iou, sel_idx.reshape(B, G, LANES), amin, gt_bboxes, lab, pad_bbox_flag)

    weights = jnp.ones((B, N), dtype=gt_bboxes.dtype)
    bboxes = jnp.transpose(bboxes_t, (0, 2, 1))
    return labels[:, 0], weights, bboxes, metrics[:, 0]


# speedup vs baseline: 2.4846x; 1.0564x over previous
import functools

import jax
import jax.numpy as jnp
from jax import lax
from jax.experimental import pallas as pl
from jax.experimental.pallas import tpu as pltpu
from jax.experimental.pallas import tpu_sc as plsc

NUM_CLASSES = 80
SOFT_CENTER_RADIUS = 3.0
TOPK = 13
IOU_WEIGHT = 3.0
INF = 100000000.0
EPS = 1e-7
BIG = 3.0e38
LANES = 16


def _cost_kernel(pb_ref, ps_ref, pr_ref, gt_ref, lab_ref, flag_ref,
                 cost_out, iou_out, amin_out):
    N = pb_ref.shape[2]
    G = gt_ref.shape[1]

    pb = pb_ref[0]
    pr = pr_ref[...]
    gt = gt_ref[0]
    lab = lab_ref[0]
    flag = flag_ref[0]

    px = pr[0:1, :]
    py = pr[1:2, :]
    pstride = pr[2:3, :]
    x1 = pb[0:1, :]
    y1 = pb[1:2, :]
    x2 = pb[2:3, :]
    y2 = pb[3:4, :]
    gx1 = gt[:, 0:1]
    gy1 = gt[:, 1:2]
    gx2 = gt[:, 2:3]
    gy2 = gt[:, 3:4]

    in_gts = (px > gx1) & (py > gy1) & (px < gx2) & (py < gy2) & (flag > 0)
    valid = jnp.sum(in_gts.astype(jnp.float32), axis=0, keepdims=True) > 0
    validf = valid.astype(jnp.float32)

    gcx = (gx1 + gx2) * 0.5
    gcy = (gy1 + gy2) * 0.5
    dist = jnp.sqrt((px - gcx) ** 2 + (py - gcy) ** 2) / pstride
    dist = dist * validf
    soft = jnp.power(10.0, dist - SOFT_CENTER_RADIUS)

    iw = jnp.maximum(jnp.minimum(x2, gx2) - jnp.maximum(x1, gx1), 0.0)
    ih = jnp.maximum(jnp.minimum(y2, gy2) - jnp.maximum(y1, gy1), 0.0)
    overlap = iw * ih
    area1 = (x2 - x1) * (y2 - y1)
    area2 = (gx2 - gx1) * (gy2 - gy1)
    union = jnp.maximum(area1 + area2 - overlap, 1e-6)
    iou = overlap / union
    iou_cost = -jnp.log(iou + EPS) * IOU_WEIGHT

    sc_nt = ps_ref[0]
    idx_nt = jnp.broadcast_to(jnp.transpose(lab, (1, 0)), (N, G))
    x = jnp.transpose(jnp.take_along_axis(sc_nt, idx_nt, axis=1), (1, 0))
    sig = jax.nn.sigmoid(x)
    bce = jnp.maximum(x, 0.0) - x * iou + jnp.log1p(jnp.exp(-jnp.abs(x)))
    cost = bce * (iou - sig) ** 2 + iou_cost + soft
    cost = jnp.where(valid, cost, INF)

    gidx = lax.broadcasted_iota(jnp.int32, (G, N), 0)
    rmin = jnp.min(cost, axis=0, keepdims=True)
    amin = jnp.min(jnp.where(cost == rmin, gidx, G), axis=0, keepdims=True)

    cost_out[0] = cost
    iou_out[0] = iou
    amin_out[0] = amin


def _sc_topk(cost_hbm, iou_hbm, idx_hbm, cost_v, iou_v, out_v,
             gm_v, best_v, bidx_v, sem):
    ncols = cost_hbm.shape[0]
    n = cost_hbm.shape[1]
    nch = n // LANES
    grp = 15
    ngrp = nch // grp
    nw = 32
    cols_per_w = ncols // nw
    wid = lax.axis_index("s") * 2 + lax.axis_index("c")
    base = wid * cols_per_w

    iota16 = lax.broadcasted_iota(jnp.int32, (LANES,), 0)
    zeros16 = jnp.zeros((LANES,), jnp.int32)
    last16 = jnp.full((LANES,), LANES - 1, jnp.int32)
    rots = [(iota16 + (1 << s)) % LANES for s in range(4)]

    def splat(vec, idx):
        return jnp.take_along_axis(vec, idx, axis=0)

    def lane_min(v):
        for r in rots:
            v = jnp.minimum(v, splat(v, r))
        return v[0]

    def lane_max(v):
        for r in rots:
            v = jnp.maximum(v, splat(v, r))
        return v[0]

    def col_body(r, carry):
        row = base + r
        cin = pltpu.make_async_copy(cost_hbm.at[row], cost_v, sem)
        iin = pltpu.make_async_copy(iou_hbm.at[row], iou_v, sem)
        cin.start()
        iin.start()
        cin.wait()
        iin.wait()

        def iou_grp(gi, c):
            m = iou_v[pl.ds(gi * grp * LANES, LANES)]
            for j in range(1, grp):
                m = jnp.maximum(m, iou_v[pl.ds((gi * grp + j) * LANES, LANES)])
            gm_v[pl.ds(gi * LANES, LANES)] = m
            return c

        lax.fori_loop(0, ngrp, iou_grp, 0)

        def tmax_merge(gi, kept):
            s = lax.sort(gm_v[pl.ds(gi * LANES, LANES)], dimension=0)
            hi = jnp.maximum(s, lax.rev(kept, (0,)))
            return lax.sort(hi, dimension=0)

        kept = lax.fori_loop(0, ngrp, tmax_merge,
                             jnp.full((LANES,), -1.0, jnp.float32))
        t_iou = kept[3]

        best_v[...] = jnp.full((LANES,), -1.0, jnp.float32)

        def iou_passb(gi, c):
            gmv = gm_v[pl.ds(gi * LANES, LANES)]

            @pl.when(lane_max(gmv) >= t_iou)
            def _():
                for j in range(grp):
                    v = iou_v[pl.ds((gi * grp + j) * LANES, LANES)]
                    s = lax.sort(v, dimension=0)
                    hi = jnp.maximum(s, lax.rev(best_v[...], (0,)))
                    best_v[...] = lax.sort(hi, dimension=0)

            return c

        lax.fori_loop(0, ngrp, iou_passb, 0)
        ibest = best_v[...]

        def sum_body(j, acc):
            return acc + splat(ibest, last16 - j)

        s13 = lax.fori_loop(0, TOPK, sum_body, jnp.zeros((LANES,), jnp.float32))
        k = jnp.maximum(s13.astype(jnp.int32), 1)

        def cost_grp(gi, c):
            m = cost_v[pl.ds(gi * grp * LANES, LANES)]
            for j in range(1, grp):
                m = jnp.minimum(m, cost_v[pl.ds((gi * grp + j) * LANES, LANES)])
            gm_v[pl.ds(gi * LANES, LANES)] = m
            return c

        lax.fori_loop(0, ngrp, cost_grp, 0)

        def tmin_merge(gi, kept):
            s = lax.sort(gm_v[pl.ds(gi * LANES, LANES)], dimension=0)
            lo = jnp.minimum(s, lax.rev(kept, (0,)))
            return lax.sort(lo, dimension=0)

        kept = lax.fori_loop(0, ngrp, tmin_merge,
                             jnp.full((LANES,), BIG, jnp.float32))
        t_cost = kept[TOPK - 1]

        best_v[...] = jnp.full((LANES,), BIG, jnp.float32)
        bidx_v[...] = jnp.full((LANES,), -1, jnp.int32)

        def cost_passb(gi, c):
            gmv = gm_v[pl.ds(gi * LANES, LANES)]

            @pl.when(lane_min(gmv) <= t_cost)
            def _():
                for j in range(grp):
                    ci = gi * grp + j
                    v = cost_v[pl.ds(ci * LANES, LANES)]
                    vidx = iota16 + ci * LANES
                    s, si = plsc.sort_key_val(v, vidx)
                    rb = lax.rev(best_v[...], (0,))
                    rbi = lax.rev(bidx_v[...], (0,))
                    take = s <= rb
                    nb, nbi = plsc.sort_key_val(jnp.where(take, s, rb),
                                                jnp.where(take, si, rbi))
                    best_v[...] = nb
                    bidx_v[...] = nbi

            return c

        lax.fori_loop(0, ngrp, cost_passb, 0)

        out_v[pl.ds(r * LANES, LANES)] = jnp.where(iota16 < k, bidx_v[...], -1)
        return carry

    lax.fori_loop(0, cols_per_w, col_body, 0)
    pltpu.sync_copy(out_v, idx_hbm.at[pl.ds(base * LANES, cols_per_w * LANES)])


def _assemble_kernel(iou_ref, idx_ref, amin_ref, gt_ref, lab_ref, flag_ref,
                     lab_out, bbox_out, met_out):
    N = iou_ref.shape[2]
    G = iou_ref.shape[1]

    iou = iou_ref[0]
    sel_idx = idx_ref[0]
    amin = amin_ref[0]
    gt = gt_ref[0]
    lab = lab_ref[0]
    flag = flag_ref[0]

    pidx = lax.broadcasted_iota(jnp.int32, (G, N), 1)
    gt_ok = flag > 0

    matching = jnp.zeros((G, N), jnp.float32)
    for j in range(TOPK):
        hit = (pidx == sel_idx[:, j:j + 1]) & gt_ok
        matching = jnp.where(hit, 1.0, matching)

    cnt = jnp.sum(matching, axis=0, keepdims=True)
    gidx = lax.broadcasted_iota(jnp.int32, (G, N), 0)
    fmatch = jnp.min(jnp.where(matching > 0, gidx, G), axis=0, keepdims=True)
    mg = jnp.where(cnt > 1, amin, fmatch)
    fg = cnt > 0

    sel = (gidx == mg).astype(jnp.float32)
    met = jnp.sum(sel * iou, axis=0, keepdims=True)
    labf = jnp.sum(sel * lab.astype(jnp.float32), axis=0, keepdims=True)
    bbox = jnp.concatenate(
        [jnp.sum(sel * gt[:, c:c + 1], axis=0, keepdims=True)
         for c in range(4)], axis=0)

    lab_out[0] = jnp.where(fg, labf.astype(jnp.int32), NUM_CLASSES)
    met_out[0] = jnp.where(fg, met, 0.0)
    bbox_out[0] = jnp.where(fg, bbox, 0.0)


def kernel(pred_bboxes, pred_scores, priors, gt_labels, gt_bboxes, pad_bbox_flag):
    B, N, _ = pred_bboxes.shape
    G = gt_bboxes.shape[1]
    pb_t = jnp.transpose(pred_bboxes, (0, 2, 1))
    pr_t = jnp.transpose(priors, (1, 0))
    lab = gt_labels.astype(jnp.int32)

    cost, iou, amin = pl.pallas_call(
        _cost_kernel,
        grid=(B,),
        in_specs=[
            pl.BlockSpec((1, 4, N), lambda b: (b, 0, 0)),
            pl.BlockSpec((1, N, NUM_CLASSES), lambda b: (b, 0, 0)),
            pl.BlockSpec((4, N), lambda b: (0, 0)),
            pl.BlockSpec((1, G, 4), lambda b: (b, 0, 0)),
            pl.BlockSpec((1, G, 1), lambda b: (b, 0, 0)),
            pl.BlockSpec((1, G, 1), lambda b: (b, 0, 0)),
        ],
        out_specs=[
            pl.BlockSpec((1, G, N), lambda b: (b, 0, 0)),
            pl.BlockSpec((1, G, N), lambda b: (b, 0, 0)),
            pl.BlockSpec((1, 1, N), lambda b: (b, 0, 0)),
        ],
        out_shape=[
            jax.ShapeDtypeStruct((B, G, N), jnp.float32),
            jax.ShapeDtypeStruct((B, G, N), jnp.float32),
            jax.ShapeDtypeStruct((B, 1, N), jnp.int32),
        ],
    )(pb_t, pred_scores, pr_t, gt_bboxes, lab, pad_bbox_flag)

    ncols = B * G
    cols_per_w = ncols // 32
    ngrp = N // LANES // 15
    sc_topk = functools.partial(
        pl.kernel,
        out_type=jax.ShapeDtypeStruct((ncols * LANES,), jnp.int32),
        mesh=plsc.VectorSubcoreMesh(core_axis_name="c", subcore_axis_name="s"),
        compiler_params=pltpu.CompilerParams(needs_layout_passes=False),
        scratch_types=[
            pltpu.VMEM((N,), jnp.float32),
            pltpu.VMEM((N,), jnp.float32),
            pltpu.VMEM((cols_per_w * LANES,), jnp.int32),
            pltpu.VMEM((ngrp * LANES,), jnp.float32),
            pltpu.VMEM((LANES,), jnp.float32),
            pltpu.VMEM((LANES,), jnp.int32),
            pltpu.SemaphoreType.DMA,
        ],
    )(_sc_topk)
    sel_idx = sc_topk(cost.reshape(B * G, N), iou.reshape(B * G, N))

    labels, bboxes_t, metrics = pl.pallas_call(
        _assemble_kernel,
        grid=(B,),
        in_specs=[
            pl.BlockSpec((1, G, N), lambda b: (b, 0, 0)),
            pl.BlockSpec((1, G, LANES), lambda b: (b, 0, 0)),
            pl.BlockSpec((1, 1, N), lambda b: (b, 0, 0)),
            pl.BlockSpec((1, G, 4), lambda b: (b, 0, 0)),
            pl.BlockSpec((1, G, 1), lambda b: (b, 0, 0)),
            pl.BlockSpec((1, G, 1), lambda b: (b, 0, 0)),
        ],
        out_specs=[
            pl.BlockSpec((1, 1, N), lambda b: (b, 0, 0)),
            pl.BlockSpec((1, 4, N), lambda b: (b, 0, 0)),
            pl.BlockSpec((1, 1, N), lambda b: (b, 0, 0)),
        ],
        out_shape=[
            jax.ShapeDtypeStruct((B, 1, N), jnp.int32),
            jax.ShapeDtypeStruct((B, 4, N), jnp.float32),
            jax.ShapeDtypeStruct((B, 1, N), jnp.float32),
        ],
    )(iou, sel_idx.reshape(B, G, LANES), amin, gt_bboxes, lab, pad_bbox_flag)

    weights = jnp.ones((B, N), dtype=gt_bboxes.dtype)
    bboxes = jnp.transpose(bboxes_t, (0, 2, 1))
    return labels[:, 0], weights, bboxes, metrics[:, 0]


# speedup vs baseline: 2.7528x; 1.1080x over previous
import functools

import jax
import jax.numpy as jnp
from jax import lax
from jax.experimental import pallas as pl
from jax.experimental.pallas import tpu as pltpu
from jax.experimental.pallas import tpu_sc as plsc

NUM_CLASSES = 80
SOFT_CENTER_RADIUS = 3.0
TOPK = 13
IOU_WEIGHT = 3.0
INF = 100000000.0
EPS = 1e-7
BIG = 3.0e38
LANES = 16


def _cost_kernel(pb_ref, ps_ref, pr_ref, gt_ref, lab_ref, flag_ref,
                 cost_out, iou_out, amin_out):
    N = pb_ref.shape[2]
    G = gt_ref.shape[1]

    pb = pb_ref[0]
    pr = pr_ref[...]
    gt = gt_ref[0]
    lab = lab_ref[0]
    flag = flag_ref[0]

    px = pr[0:1, :]
    py = pr[1:2, :]
    pstride = pr[2:3, :]
    x1 = pb[0:1, :]
    y1 = pb[1:2, :]
    x2 = pb[2:3, :]
    y2 = pb[3:4, :]
    gx1 = gt[:, 0:1]
    gy1 = gt[:, 1:2]
    gx2 = gt[:, 2:3]
    gy2 = gt[:, 3:4]

    in_gts = (px > gx1) & (py > gy1) & (px < gx2) & (py < gy2) & (flag > 0)
    valid = jnp.sum(in_gts.astype(jnp.float32), axis=0, keepdims=True) > 0
    validf = valid.astype(jnp.float32)

    gcx = (gx1 + gx2) * 0.5
    gcy = (gy1 + gy2) * 0.5
    dist = jnp.sqrt((px - gcx) ** 2 + (py - gcy) ** 2) / pstride
    dist = dist * validf
    soft = jnp.power(10.0, dist - SOFT_CENTER_RADIUS)

    iw = jnp.maximum(jnp.minimum(x2, gx2) - jnp.maximum(x1, gx1), 0.0)
    ih = jnp.maximum(jnp.minimum(y2, gy2) - jnp.maximum(y1, gy1), 0.0)
    overlap = iw * ih
    area1 = (x2 - x1) * (y2 - y1)
    area2 = (gx2 - gx1) * (gy2 - gy1)
    union = jnp.maximum(area1 + area2 - overlap, 1e-6)
    iou = overlap / union
    iou_cost = -jnp.log(iou + EPS) * IOU_WEIGHT

    sc_nt = ps_ref[0]
    idx_nt = jnp.broadcast_to(jnp.transpose(lab, (1, 0)), (N, G))
    x = jnp.transpose(jnp.take_along_axis(sc_nt, idx_nt, axis=1), (1, 0))
    sig = jax.nn.sigmoid(x)
    bce = jnp.maximum(x, 0.0) - x * iou + jnp.log1p(jnp.exp(-jnp.abs(x)))
    cost = bce * (iou - sig) ** 2 + iou_cost + soft
    cost = jnp.where(valid, cost, INF)

    gidx = lax.broadcasted_iota(jnp.int32, (G, N), 0)
    rmin = jnp.min(cost, axis=0, keepdims=True)
    amin = jnp.min(jnp.where(cost == rmin, gidx, G), axis=0, keepdims=True)

    cost_out[0] = cost
    iou_out[0] = iou
    amin_out[0] = amin


def _sc_topk(cost_hbm, iou_hbm, idx_hbm, cost_v, iou_v, out_v,
             gm_v, best_v, bidx_v, sem):
    ncols = cost_hbm.shape[0]
    n = cost_hbm.shape[1]
    nch = n // LANES
    grp = 15
    ngrp = nch // grp
    nw = 32
    cols_per_w = (ncols + nw - 1) // nw
    wid = lax.axis_index("s") * 2 + lax.axis_index("c")
    base = wid * cols_per_w

    iota16 = lax.broadcasted_iota(jnp.int32, (LANES,), 0)
    zeros16 = jnp.zeros((LANES,), jnp.int32)
    last16 = jnp.full((LANES,), LANES - 1, jnp.int32)
    rots = [(iota16 + (1 << s)) % LANES for s in range(4)]

    def splat(vec, idx):
        return jnp.take_along_axis(vec, idx, axis=0)

    def lane_min(v):
        for r in rots:
            v = jnp.minimum(v, splat(v, r))
        return v[0]

    def lane_max(v):
        for r in rots:
            v = jnp.maximum(v, splat(v, r))
        return v[0]

    def col_body(r, carry):
        row = base + r

        @pl.when(row < ncols)
        def _process():
            _do_col(r, row)

        return carry

    def _do_col(r, row):
        cin = pltpu.make_async_copy(cost_hbm.at[row], cost_v, sem)
        iin = pltpu.make_async_copy(iou_hbm.at[row], iou_v, sem)
        cin.start()
        iin.start()
        cin.wait()
        iin.wait()

        def iou_grp(gi, c):
            m = iou_v[pl.ds(gi * grp * LANES, LANES)]
            for j in range(1, grp):
                m = jnp.maximum(m, iou_v[pl.ds((gi * grp + j) * LANES, LANES)])
            gm_v[pl.ds(gi * LANES, LANES)] = m
            return c

        lax.fori_loop(0, ngrp, iou_grp, 0)

        def tmax_merge(gi, kept):
            s = lax.sort(gm_v[pl.ds(gi * LANES, LANES)], dimension=0)
            hi = jnp.maximum(s, lax.rev(kept, (0,)))
            return lax.sort(hi, dimension=0)

        kept = lax.fori_loop(0, ngrp, tmax_merge,
                             jnp.full((LANES,), -1.0, jnp.float32))
        t_iou = kept[3]

        best_v[...] = jnp.full((LANES,), -1.0, jnp.float32)

        def iou_passb(gi, c):
            gmv = gm_v[pl.ds(gi * LANES, LANES)]

            @pl.when(lane_max(gmv) >= t_iou)
            def _():
                for j in range(grp):
                    v = iou_v[pl.ds((gi * grp + j) * LANES, LANES)]
                    s = lax.sort(v, dimension=0)
                    hi = jnp.maximum(s, lax.rev(best_v[...], (0,)))
                    best_v[...] = lax.sort(hi, dimension=0)

            return c

        lax.fori_loop(0, ngrp, iou_passb, 0)
        ibest = best_v[...]

        def sum_body(j, acc):
            return acc + splat(ibest, last16 - j)

        s13 = lax.fori_loop(0, TOPK, sum_body, jnp.zeros((LANES,), jnp.float32))
        k = jnp.maximum(s13.astype(jnp.int32), 1)

        def cost_grp(gi, c):
            m = cost_v[pl.ds(gi * grp * LANES, LANES)]
            for j in range(1, grp):
                m = jnp.minimum(m, cost_v[pl.ds((gi * grp + j) * LANES, LANES)])
            gm_v[pl.ds(gi * LANES, LANES)] = m
            return c

        lax.fori_loop(0, ngrp, cost_grp, 0)

        def tmin_merge(gi, kept):
            s = lax.sort(gm_v[pl.ds(gi * LANES, LANES)], dimension=0)
            lo = jnp.minimum(s, lax.rev(kept, (0,)))
            return lax.sort(lo, dimension=0)

        kept = lax.fori_loop(0, ngrp, tmin_merge,
                             jnp.full((LANES,), BIG, jnp.float32))
        t_cost = kept[TOPK - 1]

        best_v[...] = jnp.full((LANES,), BIG, jnp.float32)
        bidx_v[...] = jnp.full((LANES,), -1, jnp.int32)

        def cost_passb(gi, c):
            gmv = gm_v[pl.ds(gi * LANES, LANES)]

            @pl.when(lane_min(gmv) <= t_cost)
            def _():
                for j in range(grp):
                    ci = gi * grp + j
                    v = cost_v[pl.ds(ci * LANES, LANES)]
                    vidx = iota16 + ci * LANES
                    s, si = plsc.sort_key_val(v, vidx)
                    rb = lax.rev(best_v[...], (0,))
                    rbi = lax.rev(bidx_v[...], (0,))
                    take = s <= rb
                    nb, nbi = plsc.sort_key_val(jnp.where(take, s, rb),
                                                jnp.where(take, si, rbi))
                    best_v[...] = nb
                    bidx_v[...] = nbi

            return c

        lax.fori_loop(0, ngrp, cost_passb, 0)

        out_v[pl.ds(r * LANES, LANES)] = jnp.where(iota16 < k, bidx_v[...], -1)

    lax.fori_loop(0, cols_per_w, col_body, 0)
    pltpu.sync_copy(out_v, idx_hbm.at[pl.ds(base * LANES, cols_per_w * LANES)])


def _assemble_kernel(iou_ref, idx_ref, amin_ref, gt_ref, lab_ref, flag_ref,
                     lab_out, bbox_out, met_out):
    N = iou_ref.shape[2]
    G = iou_ref.shape[1]

    iou = iou_ref[0]
    sel_idx = idx_ref[0]
    amin = amin_ref[0]
    gt = gt_ref[0]
    lab = lab_ref[0]
    flag = flag_ref[0]

    pidx = lax.broadcasted_iota(jnp.int32, (G, N), 1)
    gt_ok = flag > 0

    matching = jnp.zeros((G, N), jnp.float32)
    for j in range(TOPK):
        hit = (pidx == sel_idx[:, j:j + 1]) & gt_ok
        matching = jnp.where(hit, 1.0, matching)

    cnt = jnp.sum(matching, axis=0, keepdims=True)
    gidx = lax.broadcasted_iota(jnp.int32, (G, N), 0)
    fmatch = jnp.min(jnp.where(matching > 0, gidx, G), axis=0, keepdims=True)
    mg = jnp.where(cnt > 1, amin, fmatch)
    fg = cnt > 0

    sel = (gidx == mg).astype(jnp.float32)
    met = jnp.sum(sel * iou, axis=0, keepdims=True)
    labf = jnp.sum(sel * lab.astype(jnp.float32), axis=0, keepdims=True)
    bbox = jnp.concatenate(
        [jnp.sum(sel * gt[:, c:c + 1], axis=0, keepdims=True)
         for c in range(4)], axis=0)

    lab_out[0] = jnp.where(fg, labf.astype(jnp.int32), NUM_CLASSES)
    met_out[0] = jnp.where(fg, met, 0.0)
    bbox_out[0] = jnp.where(fg, bbox, 0.0)


def kernel(pred_bboxes, pred_scores, priors, gt_labels, gt_bboxes, pad_bbox_flag):
    B, N, _ = pred_bboxes.shape
    G = gt_bboxes.shape[1]
    pb_t = jnp.transpose(pred_bboxes, (0, 2, 1))
    pr_t = jnp.transpose(priors, (1, 0))
    lab = gt_labels.astype(jnp.int32)

    B2 = B // 2
    ncols = B2 * G
    nw = 32
    cols_per_w = (ncols + nw - 1) // nw
    out_rows = nw * cols_per_w
    ngrp = N // LANES // 15

    sc_topk = functools.partial(
        pl.kernel,
        out_type=jax.ShapeDtypeStruct((out_rows * LANES,), jnp.int32),
        mesh=plsc.VectorSubcoreMesh(core_axis_name="c", subcore_axis_name="s"),
        compiler_params=pltpu.CompilerParams(needs_layout_passes=False),
        scratch_types=[
            pltpu.VMEM((N,), jnp.float32),
            pltpu.VMEM((N,), jnp.float32),
            pltpu.VMEM((cols_per_w * LANES,), jnp.int32),
            pltpu.VMEM((ngrp * LANES,), jnp.float32),
            pltpu.VMEM((LANES,), jnp.float32),
            pltpu.VMEM((LANES,), jnp.int32),
            pltpu.SemaphoreType.DMA,
        ],
    )(_sc_topk)

    parts = []
    for h in range(2):
        off = h * B2
        bmap = lambda b, off=off: (b + off, 0, 0)
        cost, iou, amin = pl.pallas_call(
            _cost_kernel,
            grid=(B2,),
            in_specs=[
                pl.BlockSpec((1, 4, N), bmap),
                pl.BlockSpec((1, N, NUM_CLASSES), bmap),
                pl.BlockSpec((4, N), lambda b: (0, 0)),
                pl.BlockSpec((1, G, 4), bmap),
                pl.BlockSpec((1, G, 1), bmap),
                pl.BlockSpec((1, G, 1), bmap),
            ],
            out_specs=[
                pl.BlockSpec((1, G, N), lambda b: (b, 0, 0)),
                pl.BlockSpec((1, G, N), lambda b: (b, 0, 0)),
                pl.BlockSpec((1, 1, N), lambda b: (b, 0, 0)),
            ],
            out_shape=[
                jax.ShapeDtypeStruct((B2, G, N), jnp.float32),
                jax.ShapeDtypeStruct((B2, G, N), jnp.float32),
                jax.ShapeDtypeStruct((B2, 1, N), jnp.int32),
            ],
        )(pb_t, pred_scores, pr_t, gt_bboxes, lab, pad_bbox_flag)

        sel_idx = sc_topk(cost.reshape(ncols, N), iou.reshape(ncols, N))
        sel_idx = sel_idx[:ncols * LANES].reshape(B2, G, LANES)

        labels, bboxes_t, metrics = pl.pallas_call(
            _assemble_kernel,
            grid=(B2,),
            in_specs=[
                pl.BlockSpec((1, G, N), lambda b: (b, 0, 0)),
                pl.BlockSpec((1, G, LANES), lambda b: (b, 0, 0)),
                pl.BlockSpec((1, 1, N), lambda b: (b, 0, 0)),
                pl.BlockSpec((1, G, 4), bmap),
                pl.BlockSpec((1, G, 1), bmap),
                pl.BlockSpec((1, G, 1), bmap),
            ],
            out_specs=[
                pl.BlockSpec((1, 1, N), lambda b: (b, 0, 0)),
                pl.BlockSpec((1, 4, N), lambda b: (b, 0, 0)),
                pl.BlockSpec((1, 1, N), lambda b: (b, 0, 0)),
            ],
            out_shape=[
                jax.ShapeDtypeStruct((B2, 1, N), jnp.int32),
                jax.ShapeDtypeStruct((B2, 4, N), jnp.float32),
                jax.ShapeDtypeStruct((B2, 1, N), jnp.float32),
            ],
        )(iou, sel_idx, amin, gt_bboxes, lab, pad_bbox_flag)
        parts.append((labels, bboxes_t, metrics))

    labels = jnp.concatenate([p[0] for p in parts], axis=0)
    bboxes_t = jnp.concatenate([p[1] for p in parts], axis=0)
    metrics = jnp.concatenate([p[2] for p in parts], axis=0)

    weights = jnp.ones((B, N), dtype=gt_bboxes.dtype)
    bboxes = jnp.transpose(bboxes_t, (0, 2, 1))
    return labels[:, 0], weights, bboxes, metrics[:, 0]


# speedup vs baseline: 2.7555x; 1.0010x over previous
import functools

import jax
import jax.numpy as jnp
from jax import lax
from jax.experimental import pallas as pl
from jax.experimental.pallas import tpu as pltpu
from jax.experimental.pallas import tpu_sc as plsc

NUM_CLASSES = 80
SOFT_CENTER_RADIUS = 3.0
TOPK = 13
IOU_WEIGHT = 3.0
INF = 100000000.0
EPS = 1e-7
BIG = 3.0e38
LANES = 16


def _cost_kernel(pb_ref, ps_ref, pr_ref, gt_ref, lab_ref, flag_ref,
                 cost_out, iou_out, amin_out):
    N = pb_ref.shape[2]
    G = gt_ref.shape[1]

    pb = pb_ref[0]
    pr = pr_ref[...]
    gt = gt_ref[0]
    lab = lab_ref[0]
    flag = flag_ref[0]

    px = pr[0:1, :]
    py = pr[1:2, :]
    pstride = pr[2:3, :]
    x1 = pb[0:1, :]
    y1 = pb[1:2, :]
    x2 = pb[2:3, :]
    y2 = pb[3:4, :]
    gx1 = gt[:, 0:1]
    gy1 = gt[:, 1:2]
    gx2 = gt[:, 2:3]
    gy2 = gt[:, 3:4]

    in_gts = (px > gx1) & (py > gy1) & (px < gx2) & (py < gy2) & (flag > 0)
    valid = jnp.sum(in_gts.astype(jnp.float32), axis=0, keepdims=True) > 0
    validf = valid.astype(jnp.float32)

    gcx = (gx1 + gx2) * 0.5
    gcy = (gy1 + gy2) * 0.5
    dist = jnp.sqrt((px - gcx) ** 2 + (py - gcy) ** 2) / pstride
    dist = dist * validf
    soft = jnp.power(10.0, dist - SOFT_CENTER_RADIUS)

    iw = jnp.maximum(jnp.minimum(x2, gx2) - jnp.maximum(x1, gx1), 0.0)
    ih = jnp.maximum(jnp.minimum(y2, gy2) - jnp.maximum(y1, gy1), 0.0)
    overlap = iw * ih
    area1 = (x2 - x1) * (y2 - y1)
    area2 = (gx2 - gx1) * (gy2 - gy1)
    union = jnp.maximum(area1 + area2 - overlap, 1e-6)
    iou = overlap / union
    iou_cost = -jnp.log(iou + EPS) * IOU_WEIGHT

    sc_nt = ps_ref[0]
    idx_nt = jnp.broadcast_to(jnp.transpose(lab, (1, 0)), (N, G))
    x = jnp.transpose(jnp.take_along_axis(sc_nt, idx_nt, axis=1), (1, 0))
    sig = jax.nn.sigmoid(x)
    bce = jnp.maximum(x, 0.0) - x * iou + jnp.log1p(jnp.exp(-jnp.abs(x)))
    cost = bce * (iou - sig) ** 2 + iou_cost + soft
    cost = jnp.where(valid, cost, INF)

    gidx = lax.broadcasted_iota(jnp.int32, (G, N), 0)
    rmin = jnp.min(cost, axis=0, keepdims=True)
    amin = jnp.min(jnp.where(cost == rmin, gidx, G), axis=0, keepdims=True)

    cost_out[0] = cost
    iou_out[0] = iou
    amin_out[0] = amin


def _sc_topk(cost_hbm, iou_hbm, idx_hbm, cost_v, iou_v, out_v,
             gm_v, best_v, bidx_v, sem):
    ncols = cost_hbm.shape[0]
    n = cost_hbm.shape[1]
    nch = n // LANES
    grp = 15
    ngrp = nch // grp
    nw = 32
    cols_per_w = (ncols + nw - 1) // nw
    wid = lax.axis_index("s") * 2 + lax.axis_index("c")
    base = wid * cols_per_w

    iota16 = lax.broadcasted_iota(jnp.int32, (LANES,), 0)
    last16 = jnp.full((LANES,), LANES - 1, jnp.int32)
    rots = [(iota16 + (1 << s)) % LANES for s in range(4)]

    def splat(vec, idx):
        return jnp.take_along_axis(vec, idx, axis=0)

    def lane_min(v):
        for r in rots:
            v = jnp.minimum(v, splat(v, r))
        return v[0]

    def lane_max(v):
        for r in rots:
            v = jnp.maximum(v, splat(v, r))
        return v[0]

    def col_body(r, carry):
        row = base + r

        @pl.when(row < ncols)
        def _process():
            _do_col(r, row)

        return carry

    def _do_col(r, row):
        cin = pltpu.make_async_copy(cost_hbm.at[row], cost_v, sem)
        iin = pltpu.make_async_copy(iou_hbm.at[row], iou_v, sem)
        cin.start()
        iin.start()
        cin.wait()
        iin.wait()

        def iou_grp(gi, c):
            m = iou_v[pl.ds(gi * grp * LANES, LANES)]
            for j in range(1, grp):
                m = jnp.maximum(m, iou_v[pl.ds((gi * grp + j) * LANES, LANES)])
            gm_v[pl.ds(gi * LANES, LANES)] = m
            return c

        lax.fori_loop(0, ngrp, iou_grp, 0)

        def tmax_merge(gi, kept):
            s = lax.sort(gm_v[pl.ds(gi * LANES, LANES)], dimension=0)
            hi = jnp.maximum(s, lax.rev(kept, (0,)))
            return lax.sort(hi, dimension=0)

        kept = lax.fori_loop(0, ngrp, tmax_merge,
                             jnp.full((LANES,), -1.0, jnp.float32))
        t_iou = kept[3]

        best_v[...] = jnp.full((LANES,), -1.0, jnp.float32)

        def iou_passb(gi, c):
            gmv = gm_v[pl.ds(gi * LANES, LANES)]

            @pl.when(lane_max(gmv) >= t_iou)
            def _():
                for j in range(grp):
                    v = iou_v[pl.ds((gi * grp + j) * LANES, LANES)]
                    s = lax.sort(v, dimension=0)
                    hi = jnp.maximum(s, lax.rev(best_v[...], (0,)))
                    best_v[...] = lax.sort(hi, dimension=0)

            return c

        lax.fori_loop(0, ngrp, iou_passb, 0)
        ibest = best_v[...]

        def sum_body(j, acc):
            return acc + splat(ibest, last16 - j)

        s13 = lax.fori_loop(0, TOPK, sum_body, jnp.zeros((LANES,), jnp.float32))
        k = jnp.maximum(s13.astype(jnp.int32), 1)

        def cost_grp(gi, c):
            m = cost_v[pl.ds(gi * grp * LANES, LANES)]
            for j in range(1, grp):
                m = jnp.minimum(m, cost_v[pl.ds((gi * grp + j) * LANES, LANES)])
            gm_v[pl.ds(gi * LANES, LANES)] = m
            return c

        lax.fori_loop(0, ngrp, cost_grp, 0)

        def tmin_merge(gi, kept):
            s = lax.sort(gm_v[pl.ds(gi * LANES, LANES)], dimension=0)
            lo = jnp.minimum(s, lax.rev(kept, (0,)))
            return lax.sort(lo, dimension=0)

        kept = lax.fori_loop(0, ngrp, tmin_merge,
                             jnp.full((LANES,), BIG, jnp.float32))
        t_cost = kept[TOPK - 1]

        best_v[...] = jnp.full((LANES,), BIG, jnp.float32)
        bidx_v[...] = jnp.full((LANES,), -1, jnp.int32)

        def cost_passb(gi, c):
            gmv = gm_v[pl.ds(gi * LANES, LANES)]

            @pl.when(lane_min(gmv) <= t_cost)
            def _():
                for j in range(grp):
                    ci = gi * grp + j
                    v = cost_v[pl.ds(ci * LANES, LANES)]
                    vidx = iota16 + ci * LANES
                    s, si = plsc.sort_key_val(v, vidx)
                    rb = lax.rev(best_v[...], (0,))
                    rbi = lax.rev(bidx_v[...], (0,))
                    take = s <= rb
                    nb, nbi = plsc.sort_key_val(jnp.where(take, s, rb),
                                                jnp.where(take, si, rbi))
                    best_v[...] = nb
                    bidx_v[...] = nbi

            return c

        lax.fori_loop(0, ngrp, cost_passb, 0)

        out_v[pl.ds(r * LANES, LANES)] = jnp.where(iota16 < k, bidx_v[...], -1)

    lax.fori_loop(0, cols_per_w, col_body, 0)
    pltpu.sync_copy(out_v, idx_hbm.at[pl.ds(base * LANES, cols_per_w * LANES)])


def _assemble_kernel(iou_ref, idx_ref, amin_ref, gt_ref, lab_ref, flag_ref,
                     lab_out, bbox_out, met_out):
    N = iou_ref.shape[2]
    G = iou_ref.shape[1]

    iou = iou_ref[0]
    sel_idx = idx_ref[0]
    amin = amin_ref[0]
    gt = gt_ref[0]
    lab = lab_ref[0]
    flag = flag_ref[0]

    pidx = lax.broadcasted_iota(jnp.int32, (G, N), 1)
    gt_ok = flag > 0

    matching = jnp.zeros((G, N), jnp.float32)
    for j in range(TOPK):
        hit = (pidx == sel_idx[:, j:j + 1]) & gt_ok
        matching = jnp.where(hit, 1.0, matching)

    cnt = jnp.sum(matching, axis=0, keepdims=True)
    gidx = lax.broadcasted_iota(jnp.int32, (G, N), 0)
    fmatch = jnp.min(jnp.where(matching > 0, gidx, G), axis=0, keepdims=True)
    mg = jnp.where(cnt > 1, amin, fmatch)
    fg = cnt > 0

    sel = (gidx == mg).astype(jnp.float32)
    met = jnp.sum(sel * iou, axis=0, keepdims=True)
    labf = jnp.sum(sel * lab.astype(jnp.float32), axis=0, keepdims=True)
    bbox = jnp.concatenate(
        [jnp.sum(sel * gt[:, c:c + 1], axis=0, keepdims=True)
         for c in range(4)], axis=0)

    lab_out[0] = jnp.where(fg, labf.astype(jnp.int32), NUM_CLASSES)
    met_out[0] = jnp.where(fg, met, 0.0)
    bbox_out[0] = jnp.where(fg, bbox, 0.0)


def kernel(pred_bboxes, pred_scores, priors, gt_labels, gt_bboxes, pad_bbox_flag):
    B, N, _ = pred_bboxes.shape
    G = gt_bboxes.shape[1]
    pb_t = jnp.transpose(pred_bboxes, (0, 2, 1))
    pr_t = jnp.transpose(priors, (1, 0))
    lab = gt_labels.astype(jnp.int32)

    B2 = B // 2
    ncols = B2 * G
    nw = 32
    cols_per_w = (ncols + nw - 1) // nw
    out_rows = nw * cols_per_w
    ngrp = N // LANES // 15

    sc_topk = functools.partial(
        pl.kernel,
        out_type=jax.ShapeDtypeStruct((out_rows * LANES,), jnp.int32),
        mesh=plsc.VectorSubcoreMesh(core_axis_name="c", subcore_axis_name="s"),
        compiler_params=pltpu.CompilerParams(needs_layout_passes=False),
        scratch_types=[
            pltpu.VMEM((N,), jnp.float32),
            pltpu.VMEM((N,), jnp.float32),
            pltpu.VMEM((cols_per_w * LANES,), jnp.int32),
            pltpu.VMEM((ngrp * LANES,), jnp.float32),
            pltpu.VMEM((LANES,), jnp.float32),
            pltpu.VMEM((LANES,), jnp.int32),
            pltpu.SemaphoreType.DMA,
        ],
    )(_sc_topk)

    parts = []
    for h in range(2):
        off = h * B2
        bmap = lambda b, off=off: (b + off, 0, 0)
        cost, iou, amin = pl.pallas_call(
            _cost_kernel,
            grid=(B2,),
            in_specs=[
                pl.BlockSpec((1, 4, N), bmap),
                pl.BlockSpec((1, N, NUM_CLASSES), bmap),
                pl.BlockSpec((4, N), lambda b: (0, 0)),
                pl.BlockSpec((1, G, 4), bmap),
                pl.BlockSpec((1, G, 1), bmap),
                pl.BlockSpec((1, G, 1), bmap),
            ],
            out_specs=[
                pl.BlockSpec((1, G, N), lambda b: (b, 0, 0)),
                pl.BlockSpec((1, G, N), lambda b: (b, 0, 0)),
                pl.BlockSpec((1, 1, N), lambda b: (b, 0, 0)),
            ],
            out_shape=[
                jax.ShapeDtypeStruct((B2, G, N), jnp.float32),
                jax.ShapeDtypeStruct((B2, G, N), jnp.float32),
                jax.ShapeDtypeStruct((B2, 1, N), jnp.int32),
            ],
        )(pb_t, pred_scores, pr_t, gt_bboxes, lab, pad_bbox_flag)

        sel_idx = sc_topk(cost.reshape(ncols, N), iou.reshape(ncols, N))
        sel_idx = sel_idx[:ncols * LANES].reshape(B2, G, LANES)

        labels, bboxes_t, metrics = pl.pallas_call(
            _assemble_kernel,
            grid=(B2,),
            in_specs=[
                pl.BlockSpec((1, G, N), lambda b: (b, 0, 0)),
                pl.BlockSpec((1, G, LANES), lambda b: (b, 0, 0)),
                pl.BlockSpec((1, 1, N), lambda b: (b, 0, 0)),
                pl.BlockSpec((1, G, 4), bmap),
                pl.BlockSpec((1, G, 1), bmap),
                pl.BlockSpec((1, G, 1), bmap),
            ],
            out_specs=[
                pl.BlockSpec((1, 1, N), lambda b: (b, 0, 0)),
                pl.BlockSpec((1, 4, N), lambda b: (b, 0, 0)),
                pl.BlockSpec((1, 1, N), lambda b: (b, 0, 0)),
            ],
            out_shape=[
                jax.ShapeDtypeStruct((B2, 1, N), jnp.int32),
                jax.ShapeDtypeStruct((B2, 4, N), jnp.float32),
                jax.ShapeDtypeStruct((B2, 1, N), jnp.float32),
            ],
        )(iou, sel_idx, amin, gt_bboxes, lab, pad_bbox_flag)
        parts.append((labels, bboxes_t, metrics))

    labels = jnp.concatenate([p[0] for p in parts], axis=0)
    bboxes_t = jnp.concatenate([p[1] for p in parts], axis=0)
    metrics = jnp.concatenate([p[2] for p in parts], axis=0)

    weights = jnp.ones((B, N), dtype=gt_bboxes.dtype)
    bboxes = jnp.transpose(bboxes_t, (0, 2, 1))
    return labels[:, 0], weights, bboxes, metrics[:, 0]
